# baseline (speedup 1.0000x reference)
import functools

import jax
import jax.numpy as jnp
from jax import lax
from jax.experimental import pallas as pl
from jax.experimental.pallas import tpu as pltpu
from jax.experimental.pallas import tpu_sc as plsc

N_NODES = 50000
N_EDGES = 800000
N_GRAPHS = 512

N_PAD = 51200
E_PAD = 802816
G_PAD = 520
NTILE = 16
ROWS_PT = N_PAD // NTILE
ECH = E_PAD // 128
CH_PT = ECH // NTILE
CH_PT_HALF = ECH // (2 * NTILE)
BLK = 4
FH = 32

_mesh = plsc.VectorSubcoreMesh(core_axis_name="c", subcore_axis_name="s")
_sc_params = pltpu.CompilerParams(use_tc_tiling_on_sc=False)
f32 = jnp.float32
i32 = jnp.int32


def _fill(ref, rows, value):
    width = ref.shape[1]
    v = jnp.full((16,), value, f32)

    def body(j, _):
        for w in range(width // 16):
            ref[j, pl.ds(w * 16, 16)] = v
        return 0

    lax.fori_loop(0, rows, body, 0)


def _deg_body(col2, d0, d1, acc, cv, ob):
    c = lax.axis_index("c")
    sid = lax.axis_index("s")
    _fill(ob, 128, 0.0)
    for q in range(25):
        pltpu.sync_copy(ob, acc.at[pl.ds(sid * ROWS_PT + q * 128, 128)])
    _fill(ob, 128, 1.0)
    plsc.subcore_barrier()
    base = c * (ECH // 2) + sid * CH_PT_HALF

    def body(i, _):
        pltpu.sync_copy(col2.at[pl.ds(base + i * 4, 4)], cv)
        for k in range(4):
            pltpu.sync_copy(ob, acc.at[cv.at[k]], add=True)
        return 0

    lax.fori_loop(0, CH_PT_HALF // 4, body, 0)
    plsc.subcore_barrier()
    sl = pl.ds(sid * ROWS_PT, ROWS_PT)

    @pl.when(c == 0)
    def _():
        pltpu.sync_copy(acc.at[sl], d0.at[sl])

    @pl.when(c == 1)
    def _():
        pltpu.sync_copy(acc.at[sl], d1.at[sl])


_deg = pl.kernel(
    _deg_body,
    out_type=[jax.ShapeDtypeStruct((N_PAD, 16), f32),
              jax.ShapeDtypeStruct((N_PAD, 16), f32)],
    mesh=_mesh,
    compiler_params=_sc_params,
    scratch_types=[
        pltpu.VMEM_SHARED((N_PAD, 16), f32),
        pltpu.VMEM((4, 128), i32),
        pltpu.VMEM((128, 16), f32),
    ],
)


GRP = 28
NGRP = CH_PT // GRP
NPAIR = NGRP // 2
HGRP = GRP // 2 - 1


def _spmm_body(h0, h1, row2, col2, a0, a1, acc,
               ibr0, ibc0, ibr1, ibc1, g0, g1,
               sir0, sic0, sir1, sic1, sg0, sg1, ss0, ss1):
    c = lax.axis_index("c")
    sid = lax.axis_index("s")
    ibr = (ibr0, ibr1)
    ibc = (ibc0, ibc1)
    gb = (g0, g1)
    sg = (sg0, sg1)
    ss = (ss0, ss1)
    si = ((sir0, sic0), (sir1, sic1))

    def run(h_hbm, out_hbm):
        base = sid * CH_PT

        def fire_idx(s, gidx):
            sl = pl.ds(base + gidx * GRP, GRP)
            pltpu.async_copy(row2.at[sl], ibr[s], si[s][0])
            pltpu.async_copy(col2.at[sl], ibc[s], si[s][1])

        def wait_idx(s):
            pltpu.make_async_copy(row2.at[pl.ds(base, GRP)], ibr[s], si[s][0]).wait()
            pltpu.make_async_copy(col2.at[pl.ds(base, GRP)], ibc[s], si[s][1]).wait()

        def fire_gather(p, idxref):
            pltpu.async_copy(h_hbm.at[idxref], gb[p], sg[p])

        def wait_gather(p):
            pltpu.make_async_copy(h_hbm.at[ibr[0].at[0]], gb[p], sg[p]).wait()

        def fire_scatter(p, idxref):
            pltpu.async_copy(gb[p], acc.at[idxref], ss[p], add=True)

        def wait_scatter(p):
            pltpu.make_async_copy(gb[p], acc.at[ibc[0].at[0]], ss[p]).wait()

        _fill(g0, 128, 0.0)
        for q in range(25):
            pltpu.sync_copy(g0, acc.at[pl.ds(sid * ROWS_PT + q * 128, 128)])
        plsc.subcore_barrier()

        pltpu.sync_copy(row2.at[pl.ds(base, GRP)], ibr0)
        pltpu.sync_copy(col2.at[pl.ds(base, GRP)], ibc0)
        fire_gather(0, ibr0.at[0])

        def outer(t, _):
            for s in (0, 1):
                if s == 0:
                    @pl.when(t > 0)
                    def _():
                        wait_scatter(1)

                    fire_idx(1, 2 * t + 1)
                else:
                    wait_scatter(1)

                    @pl.when(t < NPAIR - 1)
                    def _():
                        fire_idx(0, 2 * t + 2)

                def inner(u, _):
                    wait_gather(0)

                    @pl.when(u > 0)
                    def _():
                        wait_scatter(1)

                    fire_gather(1, ibr[s].at[2 * u + 1])
                    fire_scatter(0, ibc[s].at[2 * u])
                    wait_gather(1)
                    wait_scatter(0)
                    fire_gather(0, ibr[s].at[2 * u + 2])
                    fire_scatter(1, ibc[s].at[2 * u + 1])
                    return 0

                lax.fori_loop(0, HGRP, inner, 0)

                wait_gather(0)
                wait_scatter(1)
                fire_gather(1, ibr[s].at[GRP - 1])
                fire_scatter(0, ibc[s].at[GRP - 2])
                wait_gather(1)
                wait_scatter(0)
                if s == 0:
                    wait_idx(1)
                    fire_gather(0, ibr[1].at[0])
                else:
                    @pl.when(t < NPAIR - 1)
                    def _():
                        wait_idx(0)
                        fire_gather(0, ibr[0].at[0])

                fire_scatter(1, ibc[s].at[GRP - 1])
            return 0

        lax.fori_loop(0, NPAIR, outer, 0)
        wait_scatter(1)
        plsc.subcore_barrier()
        sl = pl.ds(sid * ROWS_PT, ROWS_PT)
        pltpu.sync_copy(acc.at[sl], out_hbm.at[sl])

    @pl.when(c == 0)
    def _():
        run(h0, a0)

    @pl.when(c == 1)
    def _():
        run(h1, a1)


_spmm = pl.kernel(
    _spmm_body,
    out_type=[jax.ShapeDtypeStruct((N_PAD, FH), f32),
              jax.ShapeDtypeStruct((N_PAD, FH), f32)],
    mesh=_mesh,
    compiler_params=_sc_params,
    scratch_types=[
        pltpu.VMEM_SHARED((N_PAD, FH), f32),
        pltpu.VMEM((GRP, 128), i32),
        pltpu.VMEM((GRP, 128), i32),
        pltpu.VMEM((GRP, 128), i32),
        pltpu.VMEM((GRP, 128), i32),
        pltpu.VMEM((128, FH), f32),
        pltpu.VMEM((128, FH), f32),
        pltpu.SemaphoreType.DMA,
        pltpu.SemaphoreType.DMA,
        pltpu.SemaphoreType.DMA,
        pltpu.SemaphoreType.DMA,
        pltpu.SemaphoreType.DMA,
        pltpu.SemaphoreType.DMA,
        pltpu.SemaphoreType.DMA,
        pltpu.SemaphoreType.DMA,
    ],
)


def _pool_body(h30, h31, b2, p0, p1, cnt, accp, accc, bv, hb, ob, zb32, zb16):
    c = lax.axis_index("c")
    sid = lax.axis_index("s")
    _fill(ob, 128, 1.0)

    @pl.when(sid < 13)
    def _():
        _fill(zb32, 40, 0.0)
        pltpu.sync_copy(zb32, accp.at[pl.ds(sid * 40, 40)])

    @pl.when((c == 0) & (sid < 13))
    def _():
        _fill(zb16, 40, 0.0)
        pltpu.sync_copy(zb16, accc.at[pl.ds(sid * 40, 40)])

    plsc.subcore_barrier()

    @pl.when(c == 0)
    def _():
        def body(i, _):
            pltpu.sync_copy(b2.at[pl.ds(sid * 25 + i, 1)], bv)
            pltpu.sync_copy(h30.at[pl.ds(sid * ROWS_PT + i * 128, 128)], hb)
            pltpu.sync_copy(hb, accp.at[bv.at[0]], add=True)
            pltpu.sync_copy(ob, accc.at[bv.at[0]], add=True)
            return 0

        lax.fori_loop(0, 25, body, 0)

    @pl.when(c == 1)
    def _():
        def body(i, _):
            pltpu.sync_copy(b2.at[pl.ds(sid * 25 + i, 1)], bv)
            pltpu.sync_copy(h31.at[pl.ds(sid * ROWS_PT + i * 128, 128)], hb)
            pltpu.sync_copy(hb, accp.at[bv.at[0]], add=True)
            return 0

        lax.fori_loop(0, 25, body, 0)

    plsc.subcore_barrier()
    sl = pl.ds(sid * 40, 40)

    @pl.when((c == 0) & (sid < 13))
    def _():
        pltpu.sync_copy(accp.at[sl], p0.at[sl])
        pltpu.sync_copy(accc.at[sl], cnt.at[sl])

    @pl.when((c == 1) & (sid < 13))
    def _():
        pltpu.sync_copy(accp.at[sl], p1.at[sl])


_pool = pl.kernel(
    _pool_body,
    out_type=[jax.ShapeDtypeStruct((G_PAD, FH), f32),
              jax.ShapeDtypeStruct((G_PAD, FH), f32),
              jax.ShapeDtypeStruct((G_PAD, 16), f32)],
    mesh=_mesh,
    compiler_params=_sc_params,
    scratch_types=[
        pltpu.VMEM_SHARED((G_PAD, FH), f32),
        pltpu.VMEM_SHARED((G_PAD, 16), f32),
        pltpu.VMEM((1, 128), i32),
        pltpu.VMEM((128, FH), f32),
        pltpu.VMEM((128, 16), f32),
        pltpu.VMEM((40, FH), f32),
        pltpu.VMEM((40, 16), f32),
    ],
)


def _prep_tc(x_ref, d0_ref, d1_ref, w_ref, s_ref, hs0_ref, hs1_ref):
    deg = d0_ref[:, 0] + d1_ref[:, 0] + 1.0
    s = lax.rsqrt(deg)[:, None]
    s_ref[...] = s
    hp = jnp.dot(x_ref[...], w_ref[...], preferred_element_type=f32)
    hs = hp * s
    hs0_ref[...] = hs[:, :FH]
    hs1_ref[...] = hs[:, FH:]


def _mid_tc(a0, a1, hs0, hs1, s_ref, b_ref, w_ref, o0, o1):
    s = s_ref[...]
    t = jnp.concatenate([a0[...] + hs0[...], a1[...] + hs1[...]], axis=1)
    h = jnp.maximum(t * s + b_ref[...], 0.0)
    hs = jnp.dot(h, w_ref[...], preferred_element_type=f32) * s
    o0[...] = hs[:, :FH]
    o1[...] = hs[:, FH:]


def _last_tc(a0, a1, hs0, hs1, s_ref, b_ref, o0, o1):
    s = s_ref[...]
    t = jnp.concatenate([a0[...] + hs0[...], a1[...] + hs1[...]], axis=1)
    h = t * s + b_ref[...]
    o0[...] = h[:, :FH]
    o1[...] = h[:, FH:]


def _final_tc(p0, p1, cnt_ref, wl_ref, bl_ref, out_ref):
    sums = jnp.concatenate([p0[...], p1[...]], axis=1)[:N_GRAPHS]
    c = jnp.maximum(cnt_ref[:N_GRAPHS, 0:1], 1.0)
    z = jnp.dot(sums / c, wl_ref[...], preferred_element_type=f32) + bl_ref[...]
    out_ref[...] = 1.0 / (1.0 + jnp.exp(-z))


_RB = 512
_GRID = N_PAD // _RB


def _rows_spec(w):
    return pl.BlockSpec((_RB, w), lambda i: (i, 0))


def _full_spec(shape):
    return pl.BlockSpec(shape, lambda i: tuple(0 for _ in shape))


_prep = pl.pallas_call(
    _prep_tc,
    grid=(_GRID,),
    in_specs=[_rows_spec(32), _rows_spec(16), _rows_spec(16), _full_spec((32, 64))],
    out_specs=[_rows_spec(1), _rows_spec(FH), _rows_spec(FH)],
    out_shape=[jax.ShapeDtypeStruct((N_PAD, 1), f32),
               jax.ShapeDtypeStruct((N_PAD, FH), f32),
               jax.ShapeDtypeStruct((N_PAD, FH), f32)],
)

_mid = pl.pallas_call(
    _mid_tc,
    grid=(_GRID,),
    in_specs=[_rows_spec(FH), _rows_spec(FH), _rows_spec(FH), _rows_spec(FH),
              _rows_spec(1), _full_spec((1, 64)), _full_spec((64, 64))],
    out_specs=[_rows_spec(FH), _rows_spec(FH)],
    out_shape=[jax.ShapeDtypeStruct((N_PAD, FH), f32),
               jax.ShapeDtypeStruct((N_PAD, FH), f32)],
)

_last = pl.pallas_call(
    _last_tc,
    grid=(_GRID,),
    in_specs=[_rows_spec(FH), _rows_spec(FH), _rows_spec(FH), _rows_spec(FH),
              _rows_spec(1), _full_spec((1, 64))],
    out_specs=[_rows_spec(FH), _rows_spec(FH)],
    out_shape=[jax.ShapeDtypeStruct((N_PAD, FH), f32),
               jax.ShapeDtypeStruct((N_PAD, FH), f32)],
)

_final = pl.pallas_call(
    _final_tc,
    out_shape=jax.ShapeDtypeStruct((N_GRAPHS, 1), f32),
)


def kernel(x, edge_index, batch, W1, b1, W2, b2, W3, b3, Wl, bl):
    x_p = jnp.zeros((N_PAD, 32), f32).at[:N_NODES, :27].set(x)
    w1_p = jnp.zeros((32, 64), f32).at[:27].set(W1)
    row_p = jnp.concatenate(
        [edge_index[0], jnp.zeros((E_PAD - N_EDGES,), i32)]).reshape(ECH, 128)
    col_p = jnp.concatenate(
        [edge_index[1],
         jnp.full((E_PAD - N_EDGES,), N_PAD - 1, i32)]).reshape(ECH, 128)
    batch_p = jnp.concatenate(
        [batch, jnp.full((N_PAD - N_NODES,), N_GRAPHS, i32)]).reshape(400, 128)

    d0, d1 = _deg(col_p)
    s, hs0, hs1 = _prep(x_p, d0, d1, w1_p)
    a0, a1 = _spmm(hs0, hs1, row_p, col_p)
    hs0, hs1 = _mid(a0, a1, hs0, hs1, s, b1.reshape(1, 64), W2)
    a0, a1 = _spmm(hs0, hs1, row_p, col_p)
    hs0, hs1 = _mid(a0, a1, hs0, hs1, s, b2.reshape(1, 64), W3)
    a0, a1 = _spmm(hs0, hs1, row_p, col_p)
    h30, h31 = _last(a0, a1, hs0, hs1, s, b3.reshape(1, 64))
    p0, p1, cnt = _pool(h30, h31, batch_p)
    return _final(p0, p1, cnt, Wl, bl.reshape(1, 1))


# speedup vs baseline: 1.1297x; 1.1297x over previous
import functools

import jax
import jax.numpy as jnp
from jax import lax
from jax.experimental import pallas as pl
from jax.experimental.pallas import tpu as pltpu
from jax.experimental.pallas import tpu_sc as plsc

N_NODES = 50000
N_EDGES = 800000
N_GRAPHS = 512

N_PAD = 51200
E_PAD = 802816
G_PAD = 520
NTILE = 16
ROWS_PT = N_PAD // NTILE
ECH = E_PAD // 128
CH_PT = ECH // NTILE
CH_PT_HALF = ECH // (2 * NTILE)
BLK = 4
FH = 32

_mesh = plsc.VectorSubcoreMesh(core_axis_name="c", subcore_axis_name="s")
_sc_params = pltpu.CompilerParams(use_tc_tiling_on_sc=False)
f32 = jnp.float32
i32 = jnp.int32


def _fill(ref, rows, value):
    width = ref.shape[1]
    v = jnp.full((16,), value, f32)

    def body(j, _):
        for w in range(width // 16):
            ref[j, pl.ds(w * 16, 16)] = v
        return 0

    lax.fori_loop(0, rows, body, 0)


def _deg_body(col2, d0, d1, acc, cv, ob):
    c = lax.axis_index("c")
    sid = lax.axis_index("s")
    _fill(ob, 128, 0.0)
    for q in range(25):
        pltpu.sync_copy(ob, acc.at[pl.ds(sid * ROWS_PT + q * 128, 128)])
    _fill(ob, 128, 1.0)
    plsc.subcore_barrier()
    base = c * (ECH // 2) + sid * CH_PT_HALF

    def body(i, _):
        pltpu.sync_copy(col2.at[pl.ds(base + i * 4, 4)], cv)
        for k in range(4):
            pltpu.sync_copy(ob, acc.at[cv.at[k]], add=True)
        return 0

    lax.fori_loop(0, CH_PT_HALF // 4, body, 0)
    plsc.subcore_barrier()
    sl = pl.ds(sid * ROWS_PT, ROWS_PT)

    @pl.when(c == 0)
    def _():
        pltpu.sync_copy(acc.at[sl], d0.at[sl])

    @pl.when(c == 1)
    def _():
        pltpu.sync_copy(acc.at[sl], d1.at[sl])


_deg = pl.kernel(
    _deg_body,
    out_type=[jax.ShapeDtypeStruct((N_PAD, 16), f32),
              jax.ShapeDtypeStruct((N_PAD, 16), f32)],
    mesh=_mesh,
    compiler_params=_sc_params,
    scratch_types=[
        pltpu.VMEM_SHARED((N_PAD, 16), f32),
        pltpu.VMEM((4, 128), i32),
        pltpu.VMEM((128, 16), f32),
    ],
)


BLKS = 4


def _spmm_body(h0, h1, row2, col2, a0, a1, acc, rv, cv, g0, g1, g2, g3,
               sem_g, sem_s):
    c = lax.axis_index("c")
    sid = lax.axis_index("s")
    gb = (g0, g1, g2, g3)

    def run(h_hbm, out_hbm):
        _fill(g0, 128, 0.0)
        for q in range(25):
            pltpu.sync_copy(g0, acc.at[pl.ds(sid * ROWS_PT + q * 128, 128)])
        plsc.subcore_barrier()
        base = sid * CH_PT

        def body(b, _):
            blk = base + b * BLKS
            pltpu.sync_copy(row2.at[pl.ds(blk, BLKS)], rv)
            pltpu.sync_copy(col2.at[pl.ds(blk, BLKS)], cv)
            gets = [pltpu.async_copy(h_hbm.at[rv.at[k]], gb[k], sem_g)
                    for k in range(BLKS)]
            for d in gets:
                d.wait()
            puts = [pltpu.async_copy(gb[k], acc.at[cv.at[k]], sem_s, add=True)
                    for k in range(BLKS)]
            for d in puts:
                d.wait()
            return 0

        lax.fori_loop(0, CH_PT // BLKS, body, 0)
        plsc.subcore_barrier()
        sl = pl.ds(sid * ROWS_PT, ROWS_PT)
        pltpu.sync_copy(acc.at[sl], out_hbm.at[sl])

    @pl.when(c == 0)
    def _():
        run(h0, a0)

    @pl.when(c == 1)
    def _():
        run(h1, a1)


_spmm = pl.kernel(
    _spmm_body,
    out_type=[jax.ShapeDtypeStruct((N_PAD, FH), f32),
              jax.ShapeDtypeStruct((N_PAD, FH), f32)],
    mesh=_mesh,
    compiler_params=_sc_params,
    scratch_types=[
        pltpu.VMEM_SHARED((N_PAD, FH), f32),
        pltpu.VMEM((BLKS, 128), i32),
        pltpu.VMEM((BLKS, 128), i32),
        pltpu.VMEM((128, FH), f32),
        pltpu.VMEM((128, FH), f32),
        pltpu.VMEM((128, FH), f32),
        pltpu.VMEM((128, FH), f32),
        pltpu.SemaphoreType.DMA,
        pltpu.SemaphoreType.DMA,
    ],
)


def _pool_body(h30, h31, b2, p0, p1, cnt, accp, accc, bv, hb, ob, zb32, zb16):
    c = lax.axis_index("c")
    sid = lax.axis_index("s")
    _fill(ob, 128, 1.0)

    @pl.when(sid < 13)
    def _():
        _fill(zb32, 40, 0.0)
        pltpu.sync_copy(zb32, accp.at[pl.ds(sid * 40, 40)])

    @pl.when((c == 0) & (sid < 13))
    def _():
        _fill(zb16, 40, 0.0)
        pltpu.sync_copy(zb16, accc.at[pl.ds(sid * 40, 40)])

    plsc.subcore_barrier()

    @pl.when(c == 0)
    def _():
        def body(i, _):
            pltpu.sync_copy(b2.at[pl.ds(sid * 25 + i, 1)], bv)
            pltpu.sync_copy(h30.at[pl.ds(sid * ROWS_PT + i * 128, 128)], hb)
            pltpu.sync_copy(hb, accp.at[bv.at[0]], add=True)
            pltpu.sync_copy(ob, accc.at[bv.at[0]], add=True)
            return 0

        lax.fori_loop(0, 25, body, 0)

    @pl.when(c == 1)
    def _():
        def body(i, _):
            pltpu.sync_copy(b2.at[pl.ds(sid * 25 + i, 1)], bv)
            pltpu.sync_copy(h31.at[pl.ds(sid * ROWS_PT + i * 128, 128)], hb)
            pltpu.sync_copy(hb, accp.at[bv.at[0]], add=True)
            return 0

        lax.fori_loop(0, 25, body, 0)

    plsc.subcore_barrier()
    sl = pl.ds(sid * 40, 40)

    @pl.when((c == 0) & (sid < 13))
    def _():
        pltpu.sync_copy(accp.at[sl], p0.at[sl])
        pltpu.sync_copy(accc.at[sl], cnt.at[sl])

    @pl.when((c == 1) & (sid < 13))
    def _():
        pltpu.sync_copy(accp.at[sl], p1.at[sl])


_pool = pl.kernel(
    _pool_body,
    out_type=[jax.ShapeDtypeStruct((G_PAD, FH), f32),
              jax.ShapeDtypeStruct((G_PAD, FH), f32),
              jax.ShapeDtypeStruct((G_PAD, 16), f32)],
    mesh=_mesh,
    compiler_params=_sc_params,
    scratch_types=[
        pltpu.VMEM_SHARED((G_PAD, FH), f32),
        pltpu.VMEM_SHARED((G_PAD, 16), f32),
        pltpu.VMEM((1, 128), i32),
        pltpu.VMEM((128, FH), f32),
        pltpu.VMEM((128, 16), f32),
        pltpu.VMEM((40, FH), f32),
        pltpu.VMEM((40, 16), f32),
    ],
)


def _prep_tc(x_ref, d0_ref, d1_ref, w_ref, s_ref, hs0_ref, hs1_ref):
    deg = d0_ref[:, 0] + d1_ref[:, 0] + 1.0
    s = lax.rsqrt(deg)[:, None]
    s_ref[...] = s
    hp = jnp.dot(x_ref[...], w_ref[...], preferred_element_type=f32)
    hs = hp * s
    hs0_ref[...] = hs[:, :FH]
    hs1_ref[...] = hs[:, FH:]


def _mid_tc(a0, a1, hs0, hs1, s_ref, b_ref, w_ref, o0, o1):
    s = s_ref[...]
    t = jnp.concatenate([a0[...] + hs0[...], a1[...] + hs1[...]], axis=1)
    h = jnp.maximum(t * s + b_ref[...], 0.0)
    hs = jnp.dot(h, w_ref[...], preferred_element_type=f32) * s
    o0[...] = hs[:, :FH]
    o1[...] = hs[:, FH:]


def _last_tc(a0, a1, hs0, hs1, s_ref, b_ref, o0, o1):
    s = s_ref[...]
    t = jnp.concatenate([a0[...] + hs0[...], a1[...] + hs1[...]], axis=1)
    h = t * s + b_ref[...]
    o0[...] = h[:, :FH]
    o1[...] = h[:, FH:]


def _final_tc(p0, p1, cnt_ref, wl_ref, bl_ref, out_ref):
    sums = jnp.concatenate([p0[...], p1[...]], axis=1)[:N_GRAPHS]
    c = jnp.maximum(cnt_ref[:N_GRAPHS, 0:1], 1.0)
    z = jnp.dot(sums / c, wl_ref[...], preferred_element_type=f32) + bl_ref[...]
    out_ref[...] = 1.0 / (1.0 + jnp.exp(-z))


_RB = 6400
_GRID = N_PAD // _RB


def _rows_spec(w):
    return pl.BlockSpec((_RB, w), lambda i: (i, 0))


def _full_spec(shape):
    return pl.BlockSpec(shape, lambda i: tuple(0 for _ in shape))


_prep = pl.pallas_call(
    _prep_tc,
    grid=(_GRID,),
    in_specs=[_rows_spec(32), _rows_spec(16), _rows_spec(16), _full_spec((32, 64))],
    out_specs=[_rows_spec(1), _rows_spec(FH), _rows_spec(FH)],
    out_shape=[jax.ShapeDtypeStruct((N_PAD, 1), f32),
               jax.ShapeDtypeStruct((N_PAD, FH), f32),
               jax.ShapeDtypeStruct((N_PAD, FH), f32)],
)

_mid = pl.pallas_call(
    _mid_tc,
    grid=(_GRID,),
    in_specs=[_rows_spec(FH), _rows_spec(FH), _rows_spec(FH), _rows_spec(FH),
              _rows_spec(1), _full_spec((1, 64)), _full_spec((64, 64))],
    out_specs=[_rows_spec(FH), _rows_spec(FH)],
    out_shape=[jax.ShapeDtypeStruct((N_PAD, FH), f32),
               jax.ShapeDtypeStruct((N_PAD, FH), f32)],
)

_last = pl.pallas_call(
    _last_tc,
    grid=(_GRID,),
    in_specs=[_rows_spec(FH), _rows_spec(FH), _rows_spec(FH), _rows_spec(FH),
              _rows_spec(1), _full_spec((1, 64))],
    out_specs=[_rows_spec(FH), _rows_spec(FH)],
    out_shape=[jax.ShapeDtypeStruct((N_PAD, FH), f32),
               jax.ShapeDtypeStruct((N_PAD, FH), f32)],
)

_final = pl.pallas_call(
    _final_tc,
    out_shape=jax.ShapeDtypeStruct((N_GRAPHS, 1), f32),
)


def kernel(x, edge_index, batch, W1, b1, W2, b2, W3, b3, Wl, bl):
    x_p = jnp.zeros((N_PAD, 32), f32).at[:N_NODES, :27].set(x)
    w1_p = jnp.zeros((32, 64), f32).at[:27].set(W1)
    row_p = jnp.concatenate(
        [edge_index[0], jnp.zeros((E_PAD - N_EDGES,), i32)]).reshape(ECH, 128)
    col_p = jnp.concatenate(
        [edge_index[1],
         jnp.full((E_PAD - N_EDGES,), N_PAD - 1, i32)]).reshape(ECH, 128)
    batch_p = jnp.concatenate(
        [batch, jnp.full((N_PAD - N_NODES,), N_GRAPHS, i32)]).reshape(400, 128)

    d0, d1 = _deg(col_p)
    s, hs0, hs1 = _prep(x_p, d0, d1, w1_p)
    a0, a1 = _spmm(hs0, hs1, row_p, col_p)
    hs0, hs1 = _mid(a0, a1, hs0, hs1, s, b1.reshape(1, 64), W2)
    a0, a1 = _spmm(hs0, hs1, row_p, col_p)
    hs0, hs1 = _mid(a0, a1, hs0, hs1, s, b2.reshape(1, 64), W3)
    a0, a1 = _spmm(hs0, hs1, row_p, col_p)
    h30, h31 = _last(a0, a1, hs0, hs1, s, b3.reshape(1, 64))
    p0, p1, cnt = _pool(h30, h31, batch_p)
    return _final(p0, p1, cnt, Wl, bl.reshape(1, 1))


# speedup vs baseline: 1.7388x; 1.5392x over previous
import jax
import jax.numpy as jnp
from jax import lax
from jax.experimental import pallas as pl
from jax.experimental.pallas import tpu as pltpu
from jax.experimental.pallas import tpu_sc as plsc

N_NODES = 50000
N_EDGES = 800000
N_GRAPHS = 512

N_PAD = 51200
G_PAD = 520
NTILE = 16
ROWS_PT = N_PAD // NTILE
ECH = N_EDGES // 128
CH_PT = 390
NREM = ECH - NTILE * CH_PT
CH_PW = 195
FH = 32
BLKS = 5

_mesh = plsc.VectorSubcoreMesh(core_axis_name="c", subcore_axis_name="s")
_sc_params = pltpu.CompilerParams(use_tc_tiling_on_sc=False)
f32 = jnp.float32
i32 = jnp.int32


def _fill(ref, rows, value):
    width = ref.shape[1]
    v = jnp.full((16,), value, f32)

    def body(j, _):
        for w in range(width // 16):
            ref[j, pl.ds(w * 16, 16)] = v
        return 0

    lax.fori_loop(0, rows, body, 0)


def _deg_body(col2, d0, d1, acc, cv, ob):
    c = lax.axis_index("c")
    sid = lax.axis_index("s")
    w = c * NTILE + sid
    _fill(ob, 128, 0.0)
    for q in range(25):
        pltpu.sync_copy(ob, acc.at[pl.ds(sid * ROWS_PT + q * 128, 128)])
    _fill(ob, 128, 1.0)
    plsc.subcore_barrier()
    base = w * CH_PW

    def body(i, _):
        pltpu.sync_copy(col2.at[pl.ds(base + i * BLKS, BLKS)], cv)
        for k in range(BLKS):
            pltpu.sync_copy(ob, acc.at[cv.at[k]], add=True)
        return 0

    lax.fori_loop(0, CH_PW // BLKS, body, 0)

    @pl.when(w < NREM)
    def _():
        pltpu.sync_copy(col2.at[pl.ds(NTILE * 2 * CH_PW + w, 1)], cv.at[pl.ds(0, 1)])
        pltpu.sync_copy(ob, acc.at[cv.at[0]], add=True)

    plsc.subcore_barrier()
    sl = pl.ds(sid * ROWS_PT, ROWS_PT)

    @pl.when(c == 0)
    def _():
        pltpu.sync_copy(acc.at[sl], d0.at[sl])

    @pl.when(c == 1)
    def _():
        pltpu.sync_copy(acc.at[sl], d1.at[sl])


_deg = pl.kernel(
    _deg_body,
    out_type=[jax.ShapeDtypeStruct((N_PAD, FH), f32),
              jax.ShapeDtypeStruct((N_PAD, FH), f32)],
    mesh=_mesh,
    compiler_params=_sc_params,
    scratch_types=[
        pltpu.VMEM_SHARED((N_PAD, FH), f32),
        pltpu.VMEM((BLKS, 128), i32),
        pltpu.VMEM((128, FH), f32),
    ],
)


def _spmm_body(h0, h1, row2, col2, a0, a1, acc, rv, cv, g0, g1, g2, g3, g4,
               sem_g, sem_s):
    c = lax.axis_index("c")
    sid = lax.axis_index("s")
    gb = (g0, g1, g2, g3, g4)

    def run(h_hbm, out_hbm):
        _fill(g0, 128, 0.0)
        for q in range(25):
            pltpu.sync_copy(g0, acc.at[pl.ds(sid * ROWS_PT + q * 128, 128)])
        plsc.subcore_barrier()
        base = sid * CH_PT

        def body(b, _):
            blk = base + b * BLKS
            pltpu.sync_copy(row2.at[pl.ds(blk, BLKS)], rv)
            pltpu.sync_copy(col2.at[pl.ds(blk, BLKS)], cv)
            gets = [pltpu.async_copy(h_hbm.at[rv.at[k]], gb[k], sem_g)
                    for k in range(BLKS)]
            for d in gets:
                d.wait()
            puts = [pltpu.async_copy(gb[k], acc.at[cv.at[k]], sem_s, add=True)
                    for k in range(BLKS)]
            for d in puts:
                d.wait()
            return 0

        lax.fori_loop(0, CH_PT // BLKS, body, 0)

        @pl.when(sid < NREM)
        def _():
            e = NTILE * CH_PT + sid
            pltpu.sync_copy(row2.at[pl.ds(e, 1)], rv.at[pl.ds(0, 1)])
            pltpu.sync_copy(col2.at[pl.ds(e, 1)], cv.at[pl.ds(0, 1)])
            pltpu.async_copy(h_hbm.at[rv.at[0]], g0, sem_g).wait()
            pltpu.async_copy(g0, acc.at[cv.at[0]], sem_s, add=True).wait()

        plsc.subcore_barrier()
        sl = pl.ds(sid * ROWS_PT, ROWS_PT)
        pltpu.sync_copy(acc.at[sl], out_hbm.at[sl])

    @pl.when(c == 0)
    def _():
        run(h0, a0)

    @pl.when(c == 1)
    def _():
        run(h1, a1)


_spmm = pl.kernel(
    _spmm_body,
    out_type=[jax.ShapeDtypeStruct((N_PAD, FH), f32),
              jax.ShapeDtypeStruct((N_PAD, FH), f32)],
    mesh=_mesh,
    compiler_params=_sc_params,
    scratch_types=[
        pltpu.VMEM_SHARED((N_PAD, FH), f32),
        pltpu.VMEM((BLKS, 128), i32),
        pltpu.VMEM((BLKS, 128), i32),
        pltpu.VMEM((128, FH), f32),
        pltpu.VMEM((128, FH), f32),
        pltpu.VMEM((128, FH), f32),
        pltpu.VMEM((128, FH), f32),
        pltpu.VMEM((128, FH), f32),
        pltpu.SemaphoreType.DMA,
        pltpu.SemaphoreType.DMA,
    ],
)


def _pool_body(h30, h31, b2, p0, p1, cnt, accp, accc, bv, hb, ob, zb32, zb16):
    c = lax.axis_index("c")
    sid = lax.axis_index("s")
    _fill(ob, 128, 1.0)

    @pl.when(sid < 13)
    def _():
        _fill(zb32, 40, 0.0)
        pltpu.sync_copy(zb32, accp.at[pl.ds(sid * 40, 40)])

    @pl.when((c == 0) & (sid < 13))
    def _():
        _fill(zb16, 40, 0.0)
        pltpu.sync_copy(zb16, accc.at[pl.ds(sid * 40, 40)])

    plsc.subcore_barrier()

    @pl.when(c == 0)
    def _():
        def body(i, _):
            pltpu.sync_copy(b2.at[pl.ds(sid * 25 + i, 1)], bv)
            pltpu.sync_copy(h30.at[pl.ds(sid * ROWS_PT + i * 128, 128)], hb)
            pltpu.sync_copy(hb, accp.at[bv.at[0]], add=True)
            pltpu.sync_copy(ob, accc.at[bv.at[0]], add=True)
            return 0

        lax.fori_loop(0, 25, body, 0)

    @pl.when(c == 1)
    def _():
        def body(i, _):
            pltpu.sync_copy(b2.at[pl.ds(sid * 25 + i, 1)], bv)
            pltpu.sync_copy(h31.at[pl.ds(sid * ROWS_PT + i * 128, 128)], hb)
            pltpu.sync_copy(hb, accp.at[bv.at[0]], add=True)
            return 0

        lax.fori_loop(0, 25, body, 0)

    plsc.subcore_barrier()
    sl = pl.ds(sid * 40, 40)

    @pl.when((c == 0) & (sid < 13))
    def _():
        pltpu.sync_copy(accp.at[sl], p0.at[sl])
        pltpu.sync_copy(accc.at[sl], cnt.at[sl])

    @pl.when((c == 1) & (sid < 13))
    def _():
        pltpu.sync_copy(accp.at[sl], p1.at[sl])


_pool = pl.kernel(
    _pool_body,
    out_type=[jax.ShapeDtypeStruct((G_PAD, FH), f32),
              jax.ShapeDtypeStruct((G_PAD, FH), f32),
              jax.ShapeDtypeStruct((G_PAD, 16), f32)],
    mesh=_mesh,
    compiler_params=_sc_params,
    scratch_types=[
        pltpu.VMEM_SHARED((G_PAD, FH), f32),
        pltpu.VMEM_SHARED((G_PAD, 16), f32),
        pltpu.VMEM((1, 128), i32),
        pltpu.VMEM((128, FH), f32),
        pltpu.VMEM((128, 16), f32),
        pltpu.VMEM((40, FH), f32),
        pltpu.VMEM((40, 16), f32),
    ],
)


_RBP = 1600
_GRID = N_PAD // 4 // _RBP


def _prep_tc(x_ref, d0_ref, d1_ref, w0_ref, w1_ref, s_ref, hs0_ref, hs1_ref):
    s = lax.rsqrt(d0_ref[...] + d1_ref[...] + 1.0)
    s_ref[...] = s
    xb = x_ref[...]
    hs0_ref[...] = jnp.dot(xb, w0_ref[...], preferred_element_type=f32) * s
    hs1_ref[...] = jnp.dot(xb, w1_ref[...], preferred_element_type=f32) * s


def _mid_tc(a0, a1, hs0, hs1, s_ref, b0_ref, b1_ref,
            w00, w01, w10, w11, o0, o1):
    s = s_ref[...]
    h0 = jnp.maximum((a0[...] + hs0[...]) * s + b0_ref[...], 0.0)
    h1 = jnp.maximum((a1[...] + hs1[...]) * s + b1_ref[...], 0.0)
    o0[...] = (jnp.dot(h0, w00[...], preferred_element_type=f32)
               + jnp.dot(h1, w10[...], preferred_element_type=f32)) * s
    o1[...] = (jnp.dot(h0, w01[...], preferred_element_type=f32)
               + jnp.dot(h1, w11[...], preferred_element_type=f32)) * s


def _last_tc(a0, a1, hs0, hs1, s_ref, b0_ref, b1_ref, o0, o1):
    s = s_ref[...]
    o0[...] = (a0[...] + hs0[...]) * s + b0_ref[...]
    o1[...] = (a1[...] + hs1[...]) * s + b1_ref[...]


def _final_tc(p0, p1, cnt_ref, wl_ref, bl_ref, out_ref):
    sums = jnp.concatenate([p0[...], p1[...]], axis=1)[:N_GRAPHS]
    c = jnp.maximum(cnt_ref[:N_GRAPHS, 0:1], 1.0)
    z = jnp.dot(sums / c, wl_ref[...], preferred_element_type=f32) + bl_ref[...]
    out_ref[...] = 1.0 / (1.0 + jnp.exp(-z))


def _pk_spec():
    return pl.BlockSpec((_RBP, 128), lambda i: (i, 0))


def _full_spec(shape):
    return pl.BlockSpec(shape, lambda i: tuple(0 for _ in shape))


_PK = jax.ShapeDtypeStruct((N_PAD // 4, 128), f32)

_prep = pl.pallas_call(
    _prep_tc,
    grid=(_GRID,),
    in_specs=[_pk_spec(), _pk_spec(), _pk_spec(),
              _full_spec((128, 128)), _full_spec((128, 128))],
    out_specs=[_pk_spec(), _pk_spec(), _pk_spec()],
    out_shape=[_PK, _PK, _PK],
)

_mid = pl.pallas_call(
    _mid_tc,
    grid=(_GRID,),
    in_specs=[_pk_spec(), _pk_spec(), _pk_spec(), _pk_spec(), _pk_spec(),
              _full_spec((1, 128)), _full_spec((1, 128)),
              _full_spec((128, 128)), _full_spec((128, 128)),
              _full_spec((128, 128)), _full_spec((128, 128))],
    out_specs=[_pk_spec(), _pk_spec()],
    out_shape=[_PK, _PK],
)

_last = pl.pallas_call(
    _last_tc,
    grid=(_GRID,),
    in_specs=[_pk_spec(), _pk_spec(), _pk_spec(), _pk_spec(), _pk_spec(),
              _full_spec((1, 128)), _full_spec((1, 128))],
    out_specs=[_pk_spec(), _pk_spec()],
    out_shape=[_PK, _PK],
)

_final = pl.pallas_call(
    _final_tc,
    out_shape=jax.ShapeDtypeStruct((N_GRAPHS, 1), f32),
)


def _bd4(w):
    k = jnp.kron(jnp.eye(4, dtype=f32), w)
    return jnp.zeros((128, 128), f32).at[:k.shape[0], :k.shape[1]].set(k)


def _tile4(b):
    return jnp.tile(b, 4).reshape(1, 128)


def kernel(x, edge_index, batch, W1, b1, W2, b2, W3, b3, Wl, bl):
    row2 = edge_index[0].reshape(ECH, 128)
    col2 = edge_index[1].reshape(ECH, 128)
    batch_p = jnp.concatenate(
        [batch, jnp.full((N_PAD - N_NODES,), N_GRAPHS, i32)]).reshape(400, 128)
    xp = jnp.pad(x.reshape(N_NODES // 4, 108), ((0, 0), (0, 20)))

    d0, d1 = _deg(col2)
    s, hs0p, hs1p = _prep(xp, d0.reshape(N_PAD // 4, 128),
                          d1.reshape(N_PAD // 4, 128),
                          _bd4(W1[:, :FH]), _bd4(W1[:, FH:]))
    pk = lambda a: a.reshape(N_PAD // 4, 128)
    un = lambda a: a.reshape(N_PAD, FH)
    a0, a1 = _spmm(un(hs0p), un(hs1p), row2, col2)
    hs0p, hs1p = _mid(pk(a0), pk(a1), hs0p, hs1p, s,
                      _tile4(b1[:FH]), _tile4(b1[FH:]),
                      _bd4(W2[:FH, :FH]), _bd4(W2[:FH, FH:]),
                      _bd4(W2[FH:, :FH]), _bd4(W2[FH:, FH:]))
    a0, a1 = _spmm(un(hs0p), un(hs1p), row2, col2)
    hs0p, hs1p = _mid(pk(a0), pk(a1), hs0p, hs1p, s,
                      _tile4(b2[:FH]), _tile4(b2[FH:]),
                      _bd4(W3[:FH, :FH]), _bd4(W3[:FH, FH:]),
                      _bd4(W3[FH:, :FH]), _bd4(W3[FH:, FH:]))
    a0, a1 = _spmm(un(hs0p), un(hs1p), row2, col2)
    h0p, h1p = _last(pk(a0), pk(a1), hs0p, hs1p, s,
                     _tile4(b3[:FH]), _tile4(b3[FH:]))
    p0, p1, cnt = _pool(un(h0p), un(h1p), batch_p)
    return _final(p0, p1, cnt, Wl, bl.reshape(1, 1))


# speedup vs baseline: 1.8475x; 1.0625x over previous
import jax
import jax.numpy as jnp
from jax import lax
from jax.experimental import pallas as pl
from jax.experimental.pallas import tpu as pltpu
from jax.experimental.pallas import tpu_sc as plsc

N_NODES = 50000
N_EDGES = 800000
N_GRAPHS = 512

N_PAD = 51200
G_PAD = 520
NTILE = 16
ROWS_PT = N_PAD // NTILE
ECH = N_EDGES // 128
CH_PT = 390
NREM = ECH - NTILE * CH_PT
CH_PW = 195
FH = 32
BLKS = 6
BLKD = 5

_mesh = plsc.VectorSubcoreMesh(core_axis_name="c", subcore_axis_name="s")
_sc_params = pltpu.CompilerParams(use_tc_tiling_on_sc=False)
f32 = jnp.float32
i32 = jnp.int32


def _fill(ref, rows, value):
    width = ref.shape[1]
    v = jnp.full((16,), value, f32)

    def body(j, _):
        for w in range(width // 16):
            ref[j, pl.ds(w * 16, 16)] = v
        return 0

    lax.fori_loop(0, rows, body, 0)


def _deg_body(col2, d0, d1, acc, cv, ob):
    c = lax.axis_index("c")
    sid = lax.axis_index("s")
    w = c * NTILE + sid
    _fill(ob, 128, 0.0)
    for q in range(25):
        pltpu.sync_copy(ob, acc.at[pl.ds(sid * ROWS_PT + q * 128, 128)])
    _fill(ob, 128, 1.0)
    plsc.subcore_barrier()
    base = w * CH_PW

    def body(i, _):
        pltpu.sync_copy(col2.at[pl.ds(base + i * BLKD, BLKD)], cv)
        for k in range(BLKD):
            pltpu.sync_copy(ob, acc.at[cv.at[k]], add=True)
        return 0

    lax.fori_loop(0, CH_PW // BLKD, body, 0)

    @pl.when(w < NREM)
    def _():
        pltpu.sync_copy(col2.at[pl.ds(NTILE * 2 * CH_PW + w, 1)], cv.at[pl.ds(0, 1)])
        pltpu.sync_copy(ob, acc.at[cv.at[0]], add=True)

    plsc.subcore_barrier()
    sl = pl.ds(sid * ROWS_PT, ROWS_PT)

    @pl.when(c == 0)
    def _():
        pltpu.sync_copy(acc.at[sl], d0.at[sl])

    @pl.when(c == 1)
    def _():
        pltpu.sync_copy(acc.at[sl], d1.at[sl])


_deg = pl.kernel(
    _deg_body,
    out_type=[jax.ShapeDtypeStruct((N_PAD, FH), f32),
              jax.ShapeDtypeStruct((N_PAD, FH), f32)],
    mesh=_mesh,
    compiler_params=_sc_params,
    scratch_types=[
        pltpu.VMEM_SHARED((N_PAD, FH), f32),
        pltpu.VMEM((BLKD, 128), i32),
        pltpu.VMEM((128, FH), f32),
    ],
)


def _spmm_body(h0, h1, row2, col2, a0, a1, acc, rv, cv, g0, g1, g2, g3, g4, g5,
               sem_g, sem_s):
    c = lax.axis_index("c")
    sid = lax.axis_index("s")
    gb = (g0, g1, g2, g3, g4, g5)

    def run(h_hbm, out_hbm):
        _fill(g0, 128, 0.0)
        for q in range(25):
            pltpu.sync_copy(g0, acc.at[pl.ds(sid * ROWS_PT + q * 128, 128)])
        plsc.subcore_barrier()
        base = sid * CH_PT

        def body(b, _):
            blk = base + b * BLKS
            pltpu.sync_copy(row2.at[pl.ds(blk, BLKS)], rv)
            pltpu.sync_copy(col2.at[pl.ds(blk, BLKS)], cv)
            gets = [pltpu.async_copy(h_hbm.at[rv.at[k]], gb[k], sem_g)
                    for k in range(BLKS)]
            for d in gets:
                d.wait()
            puts = [pltpu.async_copy(gb[k], acc.at[cv.at[k]], sem_s, add=True)
                    for k in range(BLKS)]
            for d in puts:
                d.wait()
            return 0

        lax.fori_loop(0, CH_PT // BLKS, body, 0)

        @pl.when(sid < NREM)
        def _():
            e = NTILE * CH_PT + sid
            pltpu.sync_copy(row2.at[pl.ds(e, 1)], rv.at[pl.ds(0, 1)])
            pltpu.sync_copy(col2.at[pl.ds(e, 1)], cv.at[pl.ds(0, 1)])
            pltpu.async_copy(h_hbm.at[rv.at[0]], g0, sem_g).wait()
            pltpu.async_copy(g0, acc.at[cv.at[0]], sem_s, add=True).wait()

        plsc.subcore_barrier()
        sl = pl.ds(sid * ROWS_PT, ROWS_PT)
        pltpu.sync_copy(acc.at[sl], out_hbm.at[sl])

    @pl.when(c == 0)
    def _():
        run(h0, a0)

    @pl.when(c == 1)
    def _():
        run(h1, a1)


_spmm = pl.kernel(
    _spmm_body,
    out_type=[jax.ShapeDtypeStruct((N_PAD, FH), f32),
              jax.ShapeDtypeStruct((N_PAD, FH), f32)],
    mesh=_mesh,
    compiler_params=_sc_params,
    scratch_types=[
        pltpu.VMEM_SHARED((N_PAD, FH), f32),
        pltpu.VMEM((BLKS, 128), i32),
        pltpu.VMEM((BLKS, 128), i32),
        pltpu.VMEM((128, FH), f32),
        pltpu.VMEM((128, FH), f32),
        pltpu.VMEM((128, FH), f32),
        pltpu.VMEM((128, FH), f32),
        pltpu.VMEM((128, FH), f32),
        pltpu.VMEM((128, FH), f32),
        pltpu.SemaphoreType.DMA,
        pltpu.SemaphoreType.DMA,
    ],
)


def _pool_body(h30, h31, b2, p0, p1, cnt, accp, accc, bv, hb, ob, zb32, zb16):
    c = lax.axis_index("c")
    sid = lax.axis_index("s")
    _fill(ob, 128, 1.0)

    @pl.when(sid < 13)
    def _():
        _fill(zb32, 40, 0.0)
        pltpu.sync_copy(zb32, accp.at[pl.ds(sid * 40, 40)])

    @pl.when((c == 0) & (sid < 13))
    def _():
        _fill(zb16, 40, 0.0)
        pltpu.sync_copy(zb16, accc.at[pl.ds(sid * 40, 40)])

    plsc.subcore_barrier()

    @pl.when(c == 0)
    def _():
        def body(i, _):
            pltpu.sync_copy(b2.at[pl.ds(sid * 25 + i, 1)], bv)
            pltpu.sync_copy(h30.at[pl.ds(sid * ROWS_PT + i * 128, 128)], hb)
            pltpu.sync_copy(hb, accp.at[bv.at[0]], add=True)
            pltpu.sync_copy(ob, accc.at[bv.at[0]], add=True)
            return 0

        lax.fori_loop(0, 25, body, 0)

    @pl.when(c == 1)
    def _():
        def body(i, _):
            pltpu.sync_copy(b2.at[pl.ds(sid * 25 + i, 1)], bv)
            pltpu.sync_copy(h31.at[pl.ds(sid * ROWS_PT + i * 128, 128)], hb)
            pltpu.sync_copy(hb, accp.at[bv.at[0]], add=True)
            return 0

        lax.fori_loop(0, 25, body, 0)

    plsc.subcore_barrier()
    sl = pl.ds(sid * 40, 40)

    @pl.when((c == 0) & (sid < 13))
    def _():
        pltpu.sync_copy(accp.at[sl], p0.at[sl])
        pltpu.sync_copy(accc.at[sl], cnt.at[sl])

    @pl.when((c == 1) & (sid < 13))
    def _():
        pltpu.sync_copy(accp.at[sl], p1.at[sl])


_pool = pl.kernel(
    _pool_body,
    out_type=[jax.ShapeDtypeStruct((G_PAD, FH), f32),
              jax.ShapeDtypeStruct((G_PAD, FH), f32),
              jax.ShapeDtypeStruct((G_PAD, 16), f32)],
    mesh=_mesh,
    compiler_params=_sc_params,
    scratch_types=[
        pltpu.VMEM_SHARED((G_PAD, FH), f32),
        pltpu.VMEM_SHARED((G_PAD, 16), f32),
        pltpu.VMEM((1, 128), i32),
        pltpu.VMEM((128, FH), f32),
        pltpu.VMEM((128, 16), f32),
        pltpu.VMEM((40, FH), f32),
        pltpu.VMEM((40, 16), f32),
    ],
)


_RBP = 1600
_GRID = N_PAD // 4 // _RBP


def _prep_tc(x_ref, d0_ref, d1_ref, w0_ref, w1_ref, s_ref, hs0_ref, hs1_ref):
    s = lax.rsqrt(d0_ref[...] + d1_ref[...] + 1.0)
    s_ref[...] = s
    xb = x_ref[...]
    hs0_ref[...] = jnp.dot(xb, w0_ref[...], preferred_element_type=f32) * s
    hs1_ref[...] = jnp.dot(xb, w1_ref[...], preferred_element_type=f32) * s


def _mid_tc(a0, a1, hs0, hs1, s_ref, b0_ref, b1_ref,
            w00, w01, w10, w11, o0, o1):
    s = s_ref[...]
    h0 = jnp.maximum((a0[...] + hs0[...]) * s + b0_ref[...], 0.0)
    h1 = jnp.maximum((a1[...] + hs1[...]) * s + b1_ref[...], 0.0)
    o0[...] = (jnp.dot(h0, w00[...], preferred_element_type=f32)
               + jnp.dot(h1, w10[...], preferred_element_type=f32)) * s
    o1[...] = (jnp.dot(h0, w01[...], preferred_element_type=f32)
               + jnp.dot(h1, w11[...], preferred_element_type=f32)) * s


def _last_tc(a0, a1, hs0, hs1, s_ref, b0_ref, b1_ref, o0, o1):
    s = s_ref[...]
    o0[...] = (a0[...] + hs0[...]) * s + b0_ref[...]
    o1[...] = (a1[...] + hs1[...]) * s + b1_ref[...]


def _final_tc(p0, p1, cnt_ref, wl_ref, bl_ref, out_ref):
    sums = jnp.concatenate([p0[...], p1[...]], axis=1)[:N_GRAPHS]
    c = jnp.maximum(cnt_ref[:N_GRAPHS, 0:1], 1.0)
    z = jnp.dot(sums / c, wl_ref[...], preferred_element_type=f32) + bl_ref[...]
    out_ref[...] = 1.0 / (1.0 + jnp.exp(-z))


def _pk_spec():
    return pl.BlockSpec((_RBP, 128), lambda i: (i, 0))


def _full_spec(shape):
    return pl.BlockSpec(shape, lambda i: tuple(0 for _ in shape))


_PK = jax.ShapeDtypeStruct((N_PAD // 4, 128), f32)

_prep = pl.pallas_call(
    _prep_tc,
    grid=(_GRID,),
    in_specs=[_pk_spec(), _pk_spec(), _pk_spec(),
              _full_spec((128, 128)), _full_spec((128, 128))],
    out_specs=[_pk_spec(), _pk_spec(), _pk_spec()],
    out_shape=[_PK, _PK, _PK],
)

_mid = pl.pallas_call(
    _mid_tc,
    grid=(_GRID,),
    in_specs=[_pk_spec(), _pk_spec(), _pk_spec(), _pk_spec(), _pk_spec(),
              _full_spec((1, 128)), _full_spec((1, 128)),
              _full_spec((128, 128)), _full_spec((128, 128)),
              _full_spec((128, 128)), _full_spec((128, 128))],
    out_specs=[_pk_spec(), _pk_spec()],
    out_shape=[_PK, _PK],
)

_last = pl.pallas_call(
    _last_tc,
    grid=(_GRID,),
    in_specs=[_pk_spec(), _pk_spec(), _pk_spec(), _pk_spec(), _pk_spec(),
              _full_spec((1, 128)), _full_spec((1, 128))],
    out_specs=[_pk_spec(), _pk_spec()],
    out_shape=[_PK, _PK],
)

_final = pl.pallas_call(
    _final_tc,
    out_shape=jax.ShapeDtypeStruct((N_GRAPHS, 1), f32),
)


def _bd4(w):
    k = jnp.kron(jnp.eye(4, dtype=f32), w)
    return jnp.zeros((128, 128), f32).at[:k.shape[0], :k.shape[1]].set(k)


def _tile4(b):
    return jnp.tile(b, 4).reshape(1, 128)


def kernel(x, edge_index, batch, W1, b1, W2, b2, W3, b3, Wl, bl):
    row2 = edge_index[0].reshape(ECH, 128)
    col2 = edge_index[1].reshape(ECH, 128)
    batch_p = jnp.concatenate(
        [batch, jnp.full((N_PAD - N_NODES,), N_GRAPHS, i32)]).reshape(400, 128)
    xp = jnp.pad(x.reshape(N_NODES // 4, 108), ((0, 0), (0, 20)))

    d0, d1 = _deg(col2)
    s, hs0p, hs1p = _prep(xp, d0.reshape(N_PAD // 4, 128),
                          d1.reshape(N_PAD // 4, 128),
                          _bd4(W1[:, :FH]), _bd4(W1[:, FH:]))
    pk = lambda a: a.reshape(N_PAD // 4, 128)
    un = lambda a: a.reshape(N_PAD, FH)
    a0, a1 = _spmm(un(hs0p), un(hs1p), row2, col2)
    hs0p, hs1p = _mid(pk(a0), pk(a1), hs0p, hs1p, s,
                      _tile4(b1[:FH]), _tile4(b1[FH:]),
                      _bd4(W2[:FH, :FH]), _bd4(W2[:FH, FH:]),
                      _bd4(W2[FH:, :FH]), _bd4(W2[FH:, FH:]))
    a0, a1 = _spmm(un(hs0p), un(hs1p), row2, col2)
    hs0p, hs1p = _mid(pk(a0), pk(a1), hs0p, hs1p, s,
                      _tile4(b2[:FH]), _tile4(b2[FH:]),
                      _bd4(W3[:FH, :FH]), _bd4(W3[:FH, FH:]),
                      _bd4(W3[FH:, :FH]), _bd4(W3[FH:, FH:]))
    a0, a1 = _spmm(un(hs0p), un(hs1p), row2, col2)
    h0p, h1p = _last(pk(a0), pk(a1), hs0p, hs1p, s,
                     _tile4(b3[:FH]), _tile4(b3[FH:]))
    p0, p1, cnt = _pool(un(h0p), un(h1p), batch_p)
    return _final(p0, p1, cnt, Wl, bl.reshape(1, 1))
